# baseline (speedup 1.0000x reference)
import jax
import jax.numpy as jnp
from jax import lax
from jax.experimental import pallas as pl
from jax.experimental.pallas import tpu as pltpu

_NSRC = 32
_NB = 8
_H = 2048
_W = 2048
_CR = 1024
_NCHUNK = _NB * (_H // _CR)
_DEPTH = 4


def _body(sx_ref, sy_ref, x_ref, y_ref, out_ref, bufs, in_sems, out_sems):
    col = lax.broadcasted_iota(jnp.int32, (1, _W), 1)

    def chunk_slice(c):
        b, h = c // (_H // _CR), c % (_H // _CR)
        return b, h * _CR

    def start_in(c):
        b, r0 = chunk_slice(c)
        pltpu.make_async_copy(
            y_ref.at[b, pl.ds(r0, _CR)], bufs.at[c % _DEPTH],
            in_sems.at[c % _DEPTH]).start()

    def wait_in(c):
        b, r0 = chunk_slice(c)
        pltpu.make_async_copy(
            y_ref.at[b, pl.ds(r0, _CR)], bufs.at[c % _DEPTH],
            in_sems.at[c % _DEPTH]).wait()

    def start_out(c):
        b, r0 = chunk_slice(c)
        pltpu.make_async_copy(
            bufs.at[c % _DEPTH], out_ref.at[b, pl.ds(r0, _CR)],
            out_sems.at[c % _DEPTH]).start()

    def wait_out(c):
        b, r0 = chunk_slice(c)
        pltpu.make_async_copy(
            bufs.at[c % _DEPTH], out_ref.at[b, pl.ds(r0, _CR)],
            out_sems.at[c % _DEPTH]).wait()

    for c in range(_DEPTH - 1):
        start_in(c)

    for c in range(_NCHUNK):
        if c + _DEPTH - 1 < _NCHUNK:
            if c >= 1:
                wait_out(c - 1)
            start_in(c + _DEPTH - 1)
        wait_in(c)
        b, r0 = chunk_slice(c)

        def fix(i, _):
            sx = sx_ref[i]
            sy = sy_ref[i]
            in_range = jnp.logical_and(sx >= r0, sx < r0 + _CR)
            loc = jnp.clip(sx - r0, 0, _CR - 1)
            val = jnp.where(in_range, x_ref[b, i], 0.0)
            row = bufs[c % _DEPTH, pl.ds(loc, 1), :]
            bufs[c % _DEPTH, pl.ds(loc, 1), :] = row + jnp.where(
                col == sy, val, 0.0)
            return 0

        lax.fori_loop(0, _NSRC, fix, 0)
        start_out(c)

    for c in range(max(0, _NCHUNK - _DEPTH), _NCHUNK):
        wait_out(c)


def kernel(Y, X, src_x, src_y):
    return pl.pallas_call(
        _body,
        in_specs=[
            pl.BlockSpec(memory_space=pltpu.SMEM),
            pl.BlockSpec(memory_space=pltpu.SMEM),
            pl.BlockSpec(memory_space=pltpu.SMEM),
            pl.BlockSpec(memory_space=pltpu.MemorySpace.HBM),
        ],
        out_specs=pl.BlockSpec(memory_space=pltpu.MemorySpace.HBM),
        out_shape=jax.ShapeDtypeStruct(Y.shape, Y.dtype),
        scratch_shapes=[
            pltpu.VMEM((_DEPTH, _CR, _W), jnp.float32),
            pltpu.SemaphoreType.DMA((_DEPTH,)),
            pltpu.SemaphoreType.DMA((_DEPTH,)),
        ],
        compiler_params=pltpu.CompilerParams(
            vmem_limit_bytes=50 * 1024 * 1024,
        ),
    )(src_x, src_y, X, Y)
